# baseline (speedup 1.0000x reference)
import functools

import jax
import jax.numpy as jnp
from jax import lax
from jax.experimental import pallas as pl
from jax.experimental.pallas import tpu as pltpu, tpu_sc as plsc

N = 10000
E = 320000
D = 128

NC, NS = 2, 16
NW = NC * NS
EPW = E // NW
K = 80
NCH = EPW // K
CP = 80
NZC = N // CP
MAXZ = -(-NZC // NS)

_mesh = plsc.VectorSubcoreMesh(core_axis_name="c", subcore_axis_name="s")


def _seg_body(g_hbm, src_hbm, dst_hbm, acc_out,
              acc_sh, idx_s, idx_d, rows0, rows1, sg0, sg1, ss0, ss1):
    c = lax.axis_index("c")
    s = lax.axis_index("s")
    wid = s * NC + c

    def _zrow(i, _):
        r = i // 8
        q = i % 8
        rows0[r, pl.ds(q * 16, 16)] = jnp.zeros((16,), jnp.float32)
        return 0
    lax.fori_loop(0, CP * 8, _zrow, 0)

    for m in range(MAXZ):
        ch = m * NS + s

        @pl.when(ch < NZC)
        def _():
            pltpu.sync_copy(rows0.at[pl.ds(0, CP)], acc_sh.at[pl.ds(ch * CP, CP)])

    plsc.subcore_barrier()

    pltpu.sync_copy(src_hbm.at[wid], idx_s)
    pltpu.sync_copy(dst_hbm.at[wid], idx_d)

    pltpu.async_copy(g_hbm.at[idx_s.at[pl.ds(0, K)]], rows0, sg0)
    pltpu.async_copy(g_hbm.at[idx_s.at[pl.ds(K, K)]], rows1, sg1)

    def _pair(jj, _):
        a = 2 * jj
        b = a + 1
        pltpu.make_async_copy(g_hbm.at[idx_s.at[pl.ds(a * K, K)]], rows0, sg0).wait()
        pltpu.async_copy(rows0, acc_sh.at[idx_d.at[a]], ss0, add=True)

        @pl.when(b < NCH)
        def _():
            pltpu.make_async_copy(g_hbm.at[idx_s.at[pl.ds(b * K, K)]], rows1, sg1).wait()
            pltpu.async_copy(rows1, acc_sh.at[idx_d.at[b]], ss1, add=True)

        pltpu.make_async_copy(rows0, acc_sh.at[idx_d.at[a]], ss0).wait()

        @pl.when(a + 2 < NCH)
        def _():
            pltpu.async_copy(g_hbm.at[idx_s.at[pl.ds((a + 2) * K, K)]], rows0, sg0)

        @pl.when(b < NCH)
        def _():
            pltpu.make_async_copy(rows1, acc_sh.at[idx_d.at[b]], ss1).wait()

            @pl.when(b + 2 < NCH)
            def _():
                pltpu.async_copy(g_hbm.at[idx_s.at[pl.ds((b + 2) * K, K)]], rows1, sg1)
        return 0
    lax.fori_loop(0, (NCH + 1) // 2, _pair, 0)

    plsc.subcore_barrier()

    for m in range(MAXZ):
        ch = m * NS + s

        @pl.when(ch < NZC)
        def _():
            off = ch * CP
            pltpu.sync_copy(acc_sh.at[pl.ds(off, CP)], rows0.at[pl.ds(0, CP)])
            pltpu.sync_copy(rows0.at[pl.ds(0, CP)], acc_out.at[pl.ds(c * N + off, CP)])


_seg_sum = pl.kernel(
    _seg_body,
    out_type=jax.ShapeDtypeStruct((2 * N, D), jnp.float32),
    mesh=_mesh,
    scratch_types=[
        pltpu.VMEM_SHARED((N, D), jnp.float32),
        pltpu.VMEM((EPW,), jnp.int32),
        pltpu.VMEM((NCH, K), jnp.int32),
        pltpu.VMEM((K, D), jnp.float32),
        pltpu.VMEM((K, D), jnp.float32),
        pltpu.SemaphoreType.DMA,
        pltpu.SemaphoreType.DMA,
        pltpu.SemaphoreType.DMA,
        pltpu.SemaphoreType.DMA,
    ],
)


DK = 80
DNCH = EPW // DK


def _deg_body(dst_hbm, deg_out,
              deg_sh, idx_d, ones_v):
    c = lax.axis_index("c")
    s = lax.axis_index("s")
    wid = s * NC + c

    def _fill(val, i, _):
        r = i // 8
        q = i % 8
        ones_v[r, pl.ds(q * 16, 16)] = jnp.full((16,), val, jnp.float32)
        return 0

    lax.fori_loop(0, CP * 8, functools.partial(_fill, 0.0), 0)
    for m in range(MAXZ):
        ch = m * NS + s

        @pl.when(ch < NZC)
        def _():
            pltpu.sync_copy(ones_v.at[pl.ds(0, CP)], deg_sh.at[pl.ds(ch * CP, CP)])

    lax.fori_loop(0, DK * 8, functools.partial(_fill, 1.0), 0)
    pltpu.sync_copy(dst_hbm.at[wid], idx_d)

    plsc.subcore_barrier()

    def _chunk(j, _):
        pltpu.sync_copy(ones_v, deg_sh.at[idx_d.at[j]], add=True)
        return 0
    lax.fori_loop(0, DNCH, _chunk, 0)

    plsc.subcore_barrier()

    for m in range(MAXZ):
        ch = m * NS + s

        @pl.when(ch < NZC)
        def _():
            off = ch * CP
            pltpu.sync_copy(deg_sh.at[pl.ds(off, CP)], ones_v.at[pl.ds(0, CP)])
            pltpu.sync_copy(ones_v.at[pl.ds(0, CP)], deg_out.at[pl.ds(c * N + off, CP)])


_deg_sum = pl.kernel(
    _deg_body,
    out_type=jax.ShapeDtypeStruct((2 * N, D), jnp.float32),
    mesh=_mesh,
    scratch_types=[
        pltpu.VMEM_SHARED((N, D), jnp.float32),
        pltpu.VMEM((DNCH, DK), jnp.int32),
        pltpu.VMEM((DK, D), jnp.float32),
    ],
)


_RB = 400
_GRID = N // _RB

_row_spec = pl.BlockSpec((_RB, D), lambda i: (i, 0))
_w_spec = pl.BlockSpec((D, D), lambda i: (0, 0))
_b_spec = pl.BlockSpec((1, D), lambda i: (0, 0))


def _pre_body(x_ref, wn_ref, ws_ref, g_ref, s_ref):
    x = x_ref[...]
    g_ref[...] = jnp.dot(x, wn_ref[...], preferred_element_type=jnp.float32)
    s_ref[...] = jnp.dot(x, ws_ref[...], preferred_element_type=jnp.float32)


_pre_tc = pl.pallas_call(
    _pre_body,
    grid=(_GRID,),
    in_specs=[_row_spec, _w_spec, _w_spec],
    out_specs=[_row_spec, _row_spec],
    out_shape=[jax.ShapeDtypeStruct((N, D), jnp.float32),
               jax.ShapeDtypeStruct((N, D), jnp.float32)],
)


def _layer_body(a0_ref, a1_ref, d0_ref, d1_ref, sprev_ref, b_ref,
                wn_ref, ws_ref, g_ref, s_ref):
    deg = d0_ref[...] + d1_ref[...]
    mean = (a0_ref[...] + a1_ref[...]) / jnp.maximum(deg, 1.0)
    h = jnp.maximum(mean + sprev_ref[...] + b_ref[...], 0.0)
    g_ref[...] = jnp.dot(h, wn_ref[...], preferred_element_type=jnp.float32)
    s_ref[...] = jnp.dot(h, ws_ref[...], preferred_element_type=jnp.float32)


_layer_tc = pl.pallas_call(
    _layer_body,
    grid=(_GRID,),
    in_specs=[_row_spec, _row_spec, _row_spec, _row_spec, _row_spec,
              _b_spec, _w_spec, _w_spec],
    out_specs=[_row_spec, _row_spec],
    out_shape=[jax.ShapeDtypeStruct((N, D), jnp.float32),
               jax.ShapeDtypeStruct((N, D), jnp.float32)],
)


def _final_body(a0_ref, a1_ref, d0_ref, d1_ref, sprev_ref, b_ref, o_ref):
    deg = d0_ref[...] + d1_ref[...]
    mean = (a0_ref[...] + a1_ref[...]) / jnp.maximum(deg, 1.0)
    o_ref[...] = mean + sprev_ref[...] + b_ref[...]


_final_tc = pl.pallas_call(
    _final_body,
    grid=(_GRID,),
    in_specs=[_row_spec, _row_spec, _row_spec, _row_spec, _row_spec, _b_spec],
    out_specs=_row_spec,
    out_shape=jax.ShapeDtypeStruct((N, D), jnp.float32),
)


def kernel(x, edge_index, W0n, W0s, b0, W1n, W1s, b1, W2n, W2s, b2):
    src = edge_index[0].astype(jnp.int32).reshape(NW, EPW)
    dst = edge_index[1].astype(jnp.int32).reshape(NW, NCH, K)
    dst_deg = dst.reshape(NW, DNCH, DK)
    b0r = b0.reshape(1, D)
    b1r = b1.reshape(1, D)
    b2r = b2.reshape(1, D)

    g0, s0 = _pre_tc(x, W0n, W0s)
    dg = _deg_sum(dst_deg)
    dg0, dg1 = dg[:N], dg[N:]
    a0 = _seg_sum(g0, src, dst)
    g1, s1 = _layer_tc(a0[:N], a0[N:], dg0, dg1, s0, b0r, W1n, W1s)
    a1 = _seg_sum(g1, src, dst)
    g2, s2 = _layer_tc(a1[:N], a1[N:], dg0, dg1, s1, b1r, W2n, W2s)
    a2 = _seg_sum(g2, src, dst)
    return _final_tc(a2[:N], a2[N:], dg0, dg1, s2, b2r)


# speedup vs baseline: 1.1843x; 1.1843x over previous
import functools

import jax
import jax.numpy as jnp
from jax import lax
from jax.experimental import pallas as pl
from jax.experimental.pallas import tpu as pltpu, tpu_sc as plsc

N = 10000
E = 320000
D = 128

NC, NS = 2, 16
NW = NC * NS
EPW = E // NW
K = 80
NCH = EPW // K
CP = 80
NZC = N // CP
MAXZ = -(-NZC // NS)

_mesh = plsc.VectorSubcoreMesh(core_axis_name="c", subcore_axis_name="s")


def _seg_body(g_hbm, src_hbm, dst_hbm, acc_out,
              acc_sh, idx_s, idx_d, rows0, rows1, sg0, sg1):
    c = lax.axis_index("c")
    s = lax.axis_index("s")
    wid = s * NC + c

    def _zrow(i, _):
        r = i // 8
        q = i % 8
        rows0[r, pl.ds(q * 16, 16)] = jnp.zeros((16,), jnp.float32)
        return 0
    lax.fori_loop(0, CP * 8, _zrow, 0)

    for m in range(MAXZ):
        ch = m * NS + s

        @pl.when(ch < NZC)
        def _():
            pltpu.sync_copy(rows0.at[pl.ds(0, CP)], acc_sh.at[pl.ds(ch * CP, CP)])

    plsc.subcore_barrier()

    pltpu.sync_copy(src_hbm.at[wid], idx_s)
    pltpu.sync_copy(dst_hbm.at[wid], idx_d)

    pltpu.async_copy(g_hbm.at[idx_s.at[pl.ds(0, K)]], rows0, sg0)
    pltpu.async_copy(g_hbm.at[idx_s.at[pl.ds(K, K)]], rows1, sg1)

    def _pair(jj, _):
        a = 2 * jj
        b = a + 1
        pltpu.make_async_copy(g_hbm.at[idx_s.at[pl.ds(a * K, K)]], rows0, sg0).wait()
        pltpu.sync_copy(rows0, acc_sh.at[idx_d.at[a]], add=True)

        @pl.when(a + 2 < NCH)
        def _():
            pltpu.async_copy(g_hbm.at[idx_s.at[pl.ds((a + 2) * K, K)]], rows0, sg0)

        @pl.when(b < NCH)
        def _():
            pltpu.make_async_copy(g_hbm.at[idx_s.at[pl.ds(b * K, K)]], rows1, sg1).wait()
            pltpu.sync_copy(rows1, acc_sh.at[idx_d.at[b]], add=True)

            @pl.when(b + 2 < NCH)
            def _():
                pltpu.async_copy(g_hbm.at[idx_s.at[pl.ds((b + 2) * K, K)]], rows1, sg1)
        return 0
    lax.fori_loop(0, (NCH + 1) // 2, _pair, 0)

    plsc.subcore_barrier()

    for m in range(MAXZ):
        ch = m * NS + s

        @pl.when(ch < NZC)
        def _():
            off = ch * CP
            pltpu.sync_copy(acc_sh.at[pl.ds(off, CP)], rows0.at[pl.ds(0, CP)])
            pltpu.sync_copy(rows0.at[pl.ds(0, CP)], acc_out.at[pl.ds(c * N + off, CP)])


_seg_sum = pl.kernel(
    _seg_body,
    out_type=jax.ShapeDtypeStruct((2 * N, D), jnp.float32),
    mesh=_mesh,
    scratch_types=[
        pltpu.VMEM_SHARED((N, D), jnp.float32),
        pltpu.VMEM((EPW,), jnp.int32),
        pltpu.VMEM((NCH, K), jnp.int32),
        pltpu.VMEM((K, D), jnp.float32),
        pltpu.VMEM((K, D), jnp.float32),
        pltpu.SemaphoreType.DMA,
        pltpu.SemaphoreType.DMA,
    ],
)


DK = 80
DNCH = EPW // DK


def _deg_body(dst_hbm, deg_out,
              deg_sh, idx_d, ones_v):
    c = lax.axis_index("c")
    s = lax.axis_index("s")
    wid = s * NC + c

    def _fill(val, i, _):
        r = i // 8
        q = i % 8
        ones_v[r, pl.ds(q * 16, 16)] = jnp.full((16,), val, jnp.float32)
        return 0

    lax.fori_loop(0, CP * 8, functools.partial(_fill, 0.0), 0)
    for m in range(MAXZ):
        ch = m * NS + s

        @pl.when(ch < NZC)
        def _():
            pltpu.sync_copy(ones_v.at[pl.ds(0, CP)], deg_sh.at[pl.ds(ch * CP, CP)])

    lax.fori_loop(0, DK * 8, functools.partial(_fill, 1.0), 0)
    pltpu.sync_copy(dst_hbm.at[wid], idx_d)

    plsc.subcore_barrier()

    def _chunk(j, _):
        pltpu.sync_copy(ones_v, deg_sh.at[idx_d.at[j]], add=True)
        return 0
    lax.fori_loop(0, DNCH, _chunk, 0)

    plsc.subcore_barrier()

    for m in range(MAXZ):
        ch = m * NS + s

        @pl.when(ch < NZC)
        def _():
            off = ch * CP
            pltpu.sync_copy(deg_sh.at[pl.ds(off, CP)], ones_v.at[pl.ds(0, CP)])
            pltpu.sync_copy(ones_v.at[pl.ds(0, CP)], deg_out.at[pl.ds(c * N + off, CP)])


_deg_sum = pl.kernel(
    _deg_body,
    out_type=jax.ShapeDtypeStruct((2 * N, D), jnp.float32),
    mesh=_mesh,
    scratch_types=[
        pltpu.VMEM_SHARED((N, D), jnp.float32),
        pltpu.VMEM((DNCH, DK), jnp.int32),
        pltpu.VMEM((DK, D), jnp.float32),
    ],
)


_RB = 400
_GRID = N // _RB

_row_spec = pl.BlockSpec((_RB, D), lambda i: (i, 0))
_w_spec = pl.BlockSpec((D, D), lambda i: (0, 0))
_b_spec = pl.BlockSpec((1, D), lambda i: (0, 0))


def _pre_body(x_ref, wn_ref, ws_ref, g_ref, s_ref):
    x = x_ref[...]
    g_ref[...] = jnp.dot(x, wn_ref[...], preferred_element_type=jnp.float32)
    s_ref[...] = jnp.dot(x, ws_ref[...], preferred_element_type=jnp.float32)


_pre_tc = pl.pallas_call(
    _pre_body,
    grid=(_GRID,),
    in_specs=[_row_spec, _w_spec, _w_spec],
    out_specs=[_row_spec, _row_spec],
    out_shape=[jax.ShapeDtypeStruct((N, D), jnp.float32),
               jax.ShapeDtypeStruct((N, D), jnp.float32)],
)


def _layer_body(a0_ref, a1_ref, d0_ref, d1_ref, sprev_ref, b_ref,
                wn_ref, ws_ref, g_ref, s_ref):
    deg = d0_ref[...] + d1_ref[...]
    mean = (a0_ref[...] + a1_ref[...]) / jnp.maximum(deg, 1.0)
    h = jnp.maximum(mean + sprev_ref[...] + b_ref[...], 0.0)
    g_ref[...] = jnp.dot(h, wn_ref[...], preferred_element_type=jnp.float32)
    s_ref[...] = jnp.dot(h, ws_ref[...], preferred_element_type=jnp.float32)


_layer_tc = pl.pallas_call(
    _layer_body,
    grid=(_GRID,),
    in_specs=[_row_spec, _row_spec, _row_spec, _row_spec, _row_spec,
              _b_spec, _w_spec, _w_spec],
    out_specs=[_row_spec, _row_spec],
    out_shape=[jax.ShapeDtypeStruct((N, D), jnp.float32),
               jax.ShapeDtypeStruct((N, D), jnp.float32)],
)


def _final_body(a0_ref, a1_ref, d0_ref, d1_ref, sprev_ref, b_ref, o_ref):
    deg = d0_ref[...] + d1_ref[...]
    mean = (a0_ref[...] + a1_ref[...]) / jnp.maximum(deg, 1.0)
    o_ref[...] = mean + sprev_ref[...] + b_ref[...]


_final_tc = pl.pallas_call(
    _final_body,
    grid=(_GRID,),
    in_specs=[_row_spec, _row_spec, _row_spec, _row_spec, _row_spec, _b_spec],
    out_specs=_row_spec,
    out_shape=jax.ShapeDtypeStruct((N, D), jnp.float32),
)


def kernel(x, edge_index, W0n, W0s, b0, W1n, W1s, b1, W2n, W2s, b2):
    src = edge_index[0].astype(jnp.int32).reshape(NW, EPW)
    dst = edge_index[1].astype(jnp.int32).reshape(NW, NCH, K)
    dst_deg = dst.reshape(NW, DNCH, DK)
    b0r = b0.reshape(1, D)
    b1r = b1.reshape(1, D)
    b2r = b2.reshape(1, D)

    g0, s0 = _pre_tc(x, W0n, W0s)
    dg = _deg_sum(dst_deg)
    dg0, dg1 = dg[:N], dg[N:]
    a0 = _seg_sum(g0, src, dst)
    g1, s1 = _layer_tc(a0[:N], a0[N:], dg0, dg1, s0, b0r, W1n, W1s)
    a1 = _seg_sum(g1, src, dst)
    g2, s2 = _layer_tc(a1[:N], a1[N:], dg0, dg1, s1, b1r, W2n, W2s)
    a2 = _seg_sum(g2, src, dst)
    return _final_tc(a2[:N], a2[N:], dg0, dg1, s2, b2r)


# speedup vs baseline: 1.1849x; 1.0005x over previous
import functools

import jax
import jax.numpy as jnp
from jax import lax
from jax.experimental import pallas as pl
from jax.experimental.pallas import tpu as pltpu, tpu_sc as plsc

N = 10000
E = 320000
D = 128

NC, NS = 2, 16
NW = NC * NS
EPW = E // NW
K = 80
NCH = EPW // K
CP = 80
NZC = N // CP
MAXZ = -(-NZC // NS)

_mesh = plsc.VectorSubcoreMesh(core_axis_name="c", subcore_axis_name="s")


def _seg_body(g_hbm, src_hbm, dst_hbm, acc_out,
              acc_sh, idx_s, idx_d, rows0, rows1, sg0, sg1):
    c = lax.axis_index("c")
    s = lax.axis_index("s")
    wid = s * NC + c

    def _zrow(i, _):
        r = i // 8
        q = i % 8
        rows0[r, pl.ds(q * 16, 16)] = jnp.zeros((16,), jnp.float32)
        return 0
    lax.fori_loop(0, CP * 8, _zrow, 0)

    for m in range(MAXZ):
        ch = m * NS + s

        @pl.when(ch < NZC)
        def _():
            pltpu.sync_copy(rows0.at[pl.ds(0, CP)], acc_sh.at[pl.ds(ch * CP, CP)])

    plsc.subcore_barrier()

    pltpu.sync_copy(src_hbm.at[wid], idx_s)
    pltpu.sync_copy(dst_hbm.at[wid], idx_d)

    pltpu.async_copy(g_hbm.at[idx_s.at[pl.ds(0, K)]], rows0, sg0)
    pltpu.async_copy(g_hbm.at[idx_s.at[pl.ds(K, K)]], rows1, sg1)

    def _pair(jj, _):
        a = 2 * jj
        b = a + 1
        pltpu.make_async_copy(g_hbm.at[idx_s.at[pl.ds(a * K, K)]], rows0, sg0).wait()
        pltpu.sync_copy(rows0, acc_sh.at[idx_d.at[a]], add=True)

        @pl.when(a + 2 < NCH)
        def _():
            pltpu.async_copy(g_hbm.at[idx_s.at[pl.ds((a + 2) * K, K)]], rows0, sg0)

        @pl.when(b < NCH)
        def _():
            pltpu.make_async_copy(g_hbm.at[idx_s.at[pl.ds(b * K, K)]], rows1, sg1).wait()
            pltpu.sync_copy(rows1, acc_sh.at[idx_d.at[b]], add=True)

            @pl.when(b + 2 < NCH)
            def _():
                pltpu.async_copy(g_hbm.at[idx_s.at[pl.ds((b + 2) * K, K)]], rows1, sg1)
        return 0
    lax.fori_loop(0, (NCH + 1) // 2, _pair, 0)

    plsc.subcore_barrier()

    for m in range(MAXZ):
        ch = m * NS + s

        @pl.when(ch < NZC)
        def _():
            off = ch * CP
            pltpu.sync_copy(acc_sh.at[pl.ds(off, CP)], rows0.at[pl.ds(0, CP)])
            pltpu.sync_copy(rows0.at[pl.ds(0, CP)], acc_out.at[pl.ds(c * N + off, CP)])


_seg_sum = pl.kernel(
    _seg_body,
    out_type=jax.ShapeDtypeStruct((2 * N, D), jnp.float32),
    mesh=_mesh,
    scratch_types=[
        pltpu.VMEM_SHARED((N, D), jnp.float32),
        pltpu.VMEM((EPW,), jnp.int32),
        pltpu.VMEM((NCH, K), jnp.int32),
        pltpu.VMEM((K, D), jnp.float32),
        pltpu.VMEM((K, D), jnp.float32),
        pltpu.SemaphoreType.DMA,
        pltpu.SemaphoreType.DMA,
    ],
)


DK = 80
DNCH = EPW // DK


def _deg_body(dst_hbm, deg_out,
              deg_sh, idx_d, ones_v):
    c = lax.axis_index("c")
    s = lax.axis_index("s")
    wid = s * NC + c

    def _fill(val, i, _):
        r = i // 8
        q = i % 8
        ones_v[r, pl.ds(q * 16, 16)] = jnp.full((16,), val, jnp.float32)
        return 0

    lax.fori_loop(0, CP * 8, functools.partial(_fill, 0.0), 0)
    for m in range(MAXZ):
        ch = m * NS + s

        @pl.when(ch < NZC)
        def _():
            pltpu.sync_copy(ones_v.at[pl.ds(0, CP)], deg_sh.at[pl.ds(ch * CP, CP)])

    lax.fori_loop(0, DK * 8, functools.partial(_fill, 1.0), 0)
    pltpu.sync_copy(dst_hbm.at[wid], idx_d)

    plsc.subcore_barrier()

    def _chunk(j, _):
        pltpu.sync_copy(ones_v, deg_sh.at[idx_d.at[j]], add=True)
        return 0
    lax.fori_loop(0, DNCH, _chunk, 0)

    plsc.subcore_barrier()

    for m in range(MAXZ):
        ch = m * NS + s

        @pl.when(ch < NZC)
        def _():
            off = ch * CP
            pltpu.sync_copy(deg_sh.at[pl.ds(off, CP)], ones_v.at[pl.ds(0, CP)])
            pltpu.sync_copy(ones_v.at[pl.ds(0, CP)], deg_out.at[pl.ds(c * N + off, CP)])


_deg_sum = pl.kernel(
    _deg_body,
    out_type=jax.ShapeDtypeStruct((2 * N, D), jnp.float32),
    mesh=_mesh,
    scratch_types=[
        pltpu.VMEM_SHARED((N, D), jnp.float32),
        pltpu.VMEM((DNCH, DK), jnp.int32),
        pltpu.VMEM((DK, D), jnp.float32),
    ],
)


_RB = 400
_GRID = N // _RB

_row_spec = pl.BlockSpec((_RB, D), lambda i: (i, 0))
_w_spec = pl.BlockSpec((D, D), lambda i: (0, 0))
_b_spec = pl.BlockSpec((1, D), lambda i: (0, 0))


def _pre_body(x_ref, wn_ref, ws_ref, g_ref, s_ref):
    x = x_ref[...]
    g_ref[...] = jnp.dot(x, wn_ref[...], preferred_element_type=jnp.float32)
    s_ref[...] = jnp.dot(x, ws_ref[...], preferred_element_type=jnp.float32)


_pre_tc = pl.pallas_call(
    _pre_body,
    grid=(_GRID,),
    in_specs=[_row_spec, _w_spec, _w_spec],
    out_specs=[_row_spec, _row_spec],
    out_shape=[jax.ShapeDtypeStruct((N, D), jnp.float32),
               jax.ShapeDtypeStruct((N, D), jnp.float32)],
)


def _layer_body(a0_ref, a1_ref, d0_ref, d1_ref, sprev_ref, b_ref,
                wn_ref, ws_ref, g_ref, s_ref):
    deg = d0_ref[...] + d1_ref[...]
    mean = (a0_ref[...] + a1_ref[...]) / jnp.maximum(deg, 1.0)
    h = jnp.maximum(mean + sprev_ref[...] + b_ref[...], 0.0)
    g_ref[...] = jnp.dot(h, wn_ref[...], preferred_element_type=jnp.float32)
    s_ref[...] = jnp.dot(h, ws_ref[...], preferred_element_type=jnp.float32)


_layer_tc = pl.pallas_call(
    _layer_body,
    grid=(_GRID,),
    in_specs=[_row_spec, _row_spec, _row_spec, _row_spec, _row_spec,
              _b_spec, _w_spec, _w_spec],
    out_specs=[_row_spec, _row_spec],
    out_shape=[jax.ShapeDtypeStruct((N, D), jnp.float32),
               jax.ShapeDtypeStruct((N, D), jnp.float32)],
)


def _final_body(a0_ref, a1_ref, d0_ref, d1_ref, sprev_ref, b_ref, o_ref):
    deg = d0_ref[...] + d1_ref[...]
    mean = (a0_ref[...] + a1_ref[...]) / jnp.maximum(deg, 1.0)
    o_ref[...] = mean + sprev_ref[...] + b_ref[...]


_final_tc = pl.pallas_call(
    _final_body,
    grid=(_GRID,),
    in_specs=[_row_spec, _row_spec, _row_spec, _row_spec, _row_spec, _b_spec],
    out_specs=_row_spec,
    out_shape=jax.ShapeDtypeStruct((N, D), jnp.float32),
)


def kernel(x, edge_index, W0n, W0s, b0, W1n, W1s, b1, W2n, W2s, b2):
    src = edge_index[0].astype(jnp.int32).reshape(NW, EPW)
    dst = edge_index[1].astype(jnp.int32).reshape(NW, NCH, K)
    dst_deg = dst.reshape(NW, DNCH, DK)
    b0r = b0.reshape(1, D)
    b1r = b1.reshape(1, D)
    b2r = b2.reshape(1, D)

    dg = _deg_sum(dst_deg)
    g0, s0 = _pre_tc(x, W0n, W0s)
    dg0, dg1 = dg[:N], dg[N:]
    a0 = _seg_sum(g0, src, dst)
    g1, s1 = _layer_tc(a0[:N], a0[N:], dg0, dg1, s0, b0r, W1n, W1s)
    a1 = _seg_sum(g1, src, dst)
    g2, s2 = _layer_tc(a1[:N], a1[N:], dg0, dg1, s1, b1r, W2n, W2s)
    a2 = _seg_sum(g2, src, dst)
    return _final_tc(a2[:N], a2[N:], dg0, dg1, s2, b2r)


# speedup vs baseline: 1.1996x; 1.0124x over previous
import functools

import jax
import jax.numpy as jnp
from jax import lax
from jax.experimental import pallas as pl
from jax.experimental.pallas import tpu as pltpu, tpu_sc as plsc

N = 10000
E = 320000
D = 128

NC, NS = 2, 16
NW = NC * NS
EPW = E // NW
K = 80
NCH = EPW // K
CP = 80
NZC = N // CP
MAXZ = -(-NZC // NS)

_mesh = plsc.VectorSubcoreMesh(core_axis_name="c", subcore_axis_name="s")


def _seg_body(g_hbm, src_hbm, dst_hbm, acc_out,
              acc_sh, idx_s, idx_d, rows0, rows1, sg0, sg1):
    c = lax.axis_index("c")
    s = lax.axis_index("s")
    wid = s * NC + c

    def _zrow(i, _):
        r = i // 8
        q = i % 8
        rows0[r, pl.ds(q * 16, 16)] = jnp.zeros((16,), jnp.float32)
        return 0
    lax.fori_loop(0, CP * 8, _zrow, 0)

    for m in range(MAXZ):
        ch = m * NS + s

        @pl.when(ch < NZC)
        def _():
            pltpu.sync_copy(rows0.at[pl.ds(0, CP)], acc_sh.at[pl.ds(ch * CP, CP)])

    plsc.subcore_barrier()

    pltpu.sync_copy(src_hbm.at[wid], idx_s)
    pltpu.sync_copy(dst_hbm.at[wid], idx_d)

    pltpu.async_copy(g_hbm.at[idx_s.at[pl.ds(0, K)]], rows0, sg0)
    pltpu.async_copy(g_hbm.at[idx_s.at[pl.ds(K, K)]], rows1, sg1)

    def _pair(jj, _):
        a = 2 * jj
        b = a + 1
        pltpu.make_async_copy(g_hbm.at[idx_s.at[pl.ds(a * K, K)]], rows0, sg0).wait()
        pltpu.sync_copy(rows0, acc_sh.at[idx_d.at[a]], add=True)

        @pl.when(a + 2 < NCH)
        def _():
            pltpu.async_copy(g_hbm.at[idx_s.at[pl.ds((a + 2) * K, K)]], rows0, sg0)

        @pl.when(b < NCH)
        def _():
            pltpu.make_async_copy(g_hbm.at[idx_s.at[pl.ds(b * K, K)]], rows1, sg1).wait()
            pltpu.sync_copy(rows1, acc_sh.at[idx_d.at[b]], add=True)

            @pl.when(b + 2 < NCH)
            def _():
                pltpu.async_copy(g_hbm.at[idx_s.at[pl.ds((b + 2) * K, K)]], rows1, sg1)
        return 0
    lax.fori_loop(0, (NCH + 1) // 2, _pair, 0)

    plsc.subcore_barrier()

    for m in range(MAXZ):
        ch = m * NS + s

        @pl.when(ch < NZC)
        def _():
            off = ch * CP
            pltpu.sync_copy(acc_sh.at[pl.ds(off, CP)], rows0.at[pl.ds(0, CP)])
            pltpu.sync_copy(rows0.at[pl.ds(0, CP)], acc_out.at[pl.ds(c * N + off, CP)])


_seg_sum = pl.kernel(
    _seg_body,
    out_type=jax.ShapeDtypeStruct((2 * N, D), jnp.float32),
    mesh=_mesh,
    scratch_types=[
        pltpu.VMEM_SHARED((N, D), jnp.float32),
        pltpu.VMEM((EPW,), jnp.int32),
        pltpu.VMEM((NCH, K), jnp.int32),
        pltpu.VMEM((K, D), jnp.float32),
        pltpu.VMEM((K, D), jnp.float32),
        pltpu.SemaphoreType.DMA,
        pltpu.SemaphoreType.DMA,
    ],
)


DK = 80
DNCH = EPW // DK


def _deg_body(dst_hbm, deg_out,
              deg_sh, idx_d, ones_v, sdeg):
    c = lax.axis_index("c")
    s = lax.axis_index("s")
    wid = s * NC + c

    def _fill(val, i, _):
        r = i // 8
        q = i % 8
        ones_v[r, pl.ds(q * 16, 16)] = jnp.full((16,), val, jnp.float32)
        return 0

    lax.fori_loop(0, CP * 8, functools.partial(_fill, 0.0), 0)
    for m in range(MAXZ):
        ch = m * NS + s

        @pl.when(ch < NZC)
        def _():
            pltpu.sync_copy(ones_v.at[pl.ds(0, CP)], deg_sh.at[pl.ds(ch * CP, CP)])

    lax.fori_loop(0, DK * 8, functools.partial(_fill, 1.0), 0)
    pltpu.sync_copy(dst_hbm.at[wid], idx_d)

    plsc.subcore_barrier()

    def _chunk(j, _):
        for u in range(5):
            pltpu.async_copy(ones_v, deg_sh.at[idx_d.at[5 * j + u]], sdeg)
        for u in range(5):
            pltpu.make_async_copy(ones_v, deg_sh.at[idx_d.at[5 * j + u]], sdeg).wait()
        return 0
    lax.fori_loop(0, DNCH // 5, _chunk, 0)

    plsc.subcore_barrier()

    for m in range(MAXZ):
        ch = m * NS + s

        @pl.when(ch < NZC)
        def _():
            off = ch * CP
            pltpu.sync_copy(deg_sh.at[pl.ds(off, CP)], ones_v.at[pl.ds(0, CP)])
            pltpu.sync_copy(ones_v.at[pl.ds(0, CP)], deg_out.at[pl.ds(c * N + off, CP)])


_deg_sum = pl.kernel(
    _deg_body,
    out_type=jax.ShapeDtypeStruct((2 * N, D), jnp.float32),
    mesh=_mesh,
    scratch_types=[
        pltpu.VMEM_SHARED((N, D), jnp.float32),
        pltpu.VMEM((DNCH, DK), jnp.int32),
        pltpu.VMEM((DK, D), jnp.float32),
        pltpu.SemaphoreType.DMA,
    ],
)


_RB = 400
_GRID = N // _RB

_row_spec = pl.BlockSpec((_RB, D), lambda i: (i, 0))
_w_spec = pl.BlockSpec((D, D), lambda i: (0, 0))
_b_spec = pl.BlockSpec((1, D), lambda i: (0, 0))


def _pre_body(x_ref, wn_ref, ws_ref, g_ref, s_ref):
    x = x_ref[...]
    g_ref[...] = jnp.dot(x, wn_ref[...], preferred_element_type=jnp.float32)
    s_ref[...] = jnp.dot(x, ws_ref[...], preferred_element_type=jnp.float32)


_pre_tc = pl.pallas_call(
    _pre_body,
    grid=(_GRID,),
    in_specs=[_row_spec, _w_spec, _w_spec],
    out_specs=[_row_spec, _row_spec],
    out_shape=[jax.ShapeDtypeStruct((N, D), jnp.float32),
               jax.ShapeDtypeStruct((N, D), jnp.float32)],
)


def _layer_body(a0_ref, a1_ref, d0_ref, d1_ref, sprev_ref, b_ref,
                wn_ref, ws_ref, g_ref, s_ref):
    deg = d0_ref[...] + d1_ref[...]
    mean = (a0_ref[...] + a1_ref[...]) / jnp.maximum(deg, 1.0)
    h = jnp.maximum(mean + sprev_ref[...] + b_ref[...], 0.0)
    g_ref[...] = jnp.dot(h, wn_ref[...], preferred_element_type=jnp.float32)
    s_ref[...] = jnp.dot(h, ws_ref[...], preferred_element_type=jnp.float32)


_layer_tc = pl.pallas_call(
    _layer_body,
    grid=(_GRID,),
    in_specs=[_row_spec, _row_spec, _row_spec, _row_spec, _row_spec,
              _b_spec, _w_spec, _w_spec],
    out_specs=[_row_spec, _row_spec],
    out_shape=[jax.ShapeDtypeStruct((N, D), jnp.float32),
               jax.ShapeDtypeStruct((N, D), jnp.float32)],
)


def _final_body(a0_ref, a1_ref, d0_ref, d1_ref, sprev_ref, b_ref, o_ref):
    deg = d0_ref[...] + d1_ref[...]
    mean = (a0_ref[...] + a1_ref[...]) / jnp.maximum(deg, 1.0)
    o_ref[...] = mean + sprev_ref[...] + b_ref[...]


_final_tc = pl.pallas_call(
    _final_body,
    grid=(_GRID,),
    in_specs=[_row_spec, _row_spec, _row_spec, _row_spec, _row_spec, _b_spec],
    out_specs=_row_spec,
    out_shape=jax.ShapeDtypeStruct((N, D), jnp.float32),
)


def kernel(x, edge_index, W0n, W0s, b0, W1n, W1s, b1, W2n, W2s, b2):
    src = edge_index[0].astype(jnp.int32).reshape(NW, EPW)
    dst = edge_index[1].astype(jnp.int32).reshape(NW, NCH, K)
    dst_deg = dst.reshape(NW, DNCH, DK)
    b0r = b0.reshape(1, D)
    b1r = b1.reshape(1, D)
    b2r = b2.reshape(1, D)

    dg = _deg_sum(dst_deg)
    g0, s0 = _pre_tc(x, W0n, W0s)
    dg0, dg1 = dg[:N], dg[N:]
    a0 = _seg_sum(g0, src, dst)
    g1, s1 = _layer_tc(a0[:N], a0[N:], dg0, dg1, s0, b0r, W1n, W1s)
    a1 = _seg_sum(g1, src, dst)
    g2, s2 = _layer_tc(a1[:N], a1[N:], dg0, dg1, s1, b1r, W2n, W2s)
    a2 = _seg_sum(g2, src, dst)
    return _final_tc(a2[:N], a2[N:], dg0, dg1, s2, b2r)
